# baseline (speedup 1.0000x reference)
import functools

import jax
import jax.numpy as jnp
from jax.experimental import pallas as pl
from jax.experimental.pallas import tpu as pltpu
from jax.experimental.pallas import tpu_sc as plsc

NUM_CLASSES = 117
NUM_OBJ = 80
N_DET = 1000
N_HUM = 16
POOL = 4
NPTS = POOL * POOL
C = 64
FH = FW = 64
THRESH = 0.2
P = N_HUM * N_DET
NIDX = P * NPTS
REP = 128
GATHER_WIN = 128


def _idx_body(boxes_ref, out_ref):
    h = pl.program_id(0)
    b = boxes_ref[...]
    rowi = jax.lax.broadcasted_iota(jnp.int32, (N_DET, 1), 0)
    hm = rowi == h
    bh = jnp.sum(jnp.where(hm, b, 0.0), axis=0, keepdims=True)
    x1h, y1h, x2h, y2h = (bh[:, 0:1], bh[:, 1:2], bh[:, 2:3], bh[:, 3:4])
    x1o, y1o, x2o, y2o = (b[:, 0:1], b[:, 1:2], b[:, 2:3], b[:, 3:4])
    ux1 = jnp.minimum(x1h, x1o)
    uy1 = jnp.minimum(y1h, y1o)
    ux2 = jnp.maximum(x2h, x2o)
    uy2 = jnp.maximum(y2h, y2o)
    f = (jax.lax.broadcasted_iota(jnp.int32, (1, POOL), 1).astype(jnp.float32)
         + 0.5) / POOL
    gx = ux1 + (ux2 - ux1) * f
    gy = uy1 + (uy2 - uy1) * f
    xi = jnp.clip(jnp.round(gx), 0.0, FW - 1)
    yi = jnp.clip(jnp.round(gy), 0.0, FH - 1)
    idx = jnp.concatenate(
        [yi[:, i : i + 1] * FW + xi for i in range(POOL)], axis=1
    )
    out_ref[...] = idx.astype(jnp.int32)


def _pair_indices(boxes):
    return pl.pallas_call(
        _idx_body,
        grid=(N_HUM,),
        in_specs=[pl.BlockSpec((N_DET, 4), lambda h: (0, 0))],
        out_specs=pl.BlockSpec((N_DET, NPTS), lambda h: (h, 0)),
        out_shape=jax.ShapeDtypeStruct((P, NPTS), jnp.int32),
    )(boxes)


def _sc_gather(featT, idx_flat):
    mesh = plsc.VectorSubcoreMesh(core_axis_name="c", subcore_axis_name="s")

    @functools.partial(
        pl.kernel,
        out_type=jax.ShapeDtypeStruct((NIDX, C), jnp.float32),
        mesh=mesh,
        compiler_params=pltpu.CompilerParams(use_tc_tiling_on_sc=False),
    )
    def gk(x_hbm, i_hbm, o_hbm):
        def body(i_vmem, o_vmem):
            pltpu.sync_copy(x_hbm.at[i_vmem.at[0]], o_vmem)

        nchunks = NIDX // GATHER_WIN
        pltpu.emit_pipeline(
            body,
            grid=(2, nchunks // 2),
            in_specs=[
                pl.BlockSpec(
                    (1, GATHER_WIN),
                    lambda i, j: (0, i * (nchunks // 2) + j),
                )
            ],
            out_specs=[
                pl.BlockSpec(
                    (GATHER_WIN, C),
                    lambda i, j: (i * (nchunks // 2) + j, 0),
                )
            ],
            core_axis_name=("c", "s"),
            dimension_semantics=(pltpu.PARALLEL, pltpu.PARALLEL),
        )(i_hbm, o_hbm)

    return gk(featT, idx_flat)


def _head_body(x_ref, sc_ref, lab_ref, o2t_ref, w1_ref, b1_ref, w2_ref,
               b2_ref, w3_ref, b3_ref, out_ref):
    h = pl.program_id(0)
    rowi = jax.lax.broadcasted_iota(jnp.int32, (N_DET, 1), 0)
    hm = rowi == h
    s = sc_ref[...]
    se = jnp.where(s >= THRESH, s, 0.0)
    sh = jnp.sum(jnp.where(hm, se, 0.0))
    ds = sh * se * jnp.where(hm, 0.0, 1.0)
    lab = lab_ref[...]
    l_iota = jax.lax.broadcasted_iota(jnp.int32, (N_DET, NUM_OBJ), 1).astype(
        jnp.float32)
    ohl = jnp.where(lab == l_iota, 1.0, 0.0)
    tgt = jnp.sum(ohl * o2t_ref[...], axis=1, keepdims=True)

    x = x_ref[...]
    h1 = jax.nn.relu(
        jnp.dot(x, w1_ref[...], preferred_element_type=jnp.float32)
        + b1_ref[...]
    )
    h2 = jax.nn.relu(
        jnp.dot(h1, w2_ref[...], preferred_element_type=jnp.float32)
        + b2_ref[...]
    )
    logits = (
        jnp.dot(h2, w3_ref[...], preferred_element_type=jnp.float32)
        + b3_ref[...]
    )
    k_iota = jax.lax.broadcasted_iota(jnp.int32, (N_DET, NUM_CLASSES), 1
                                      ).astype(jnp.float32)
    onehot = jnp.where(tgt == k_iota, 1.0, 0.0)
    out_ref[...] = ds * onehot * jax.nn.sigmoid(logits)


def _head(x, scores_c, labels_f, o2t_f, W1p, b1r, W2, b2r, W3, b3r):
    full = lambda shape: pl.BlockSpec(shape, lambda h: (0, 0))
    return pl.pallas_call(
        _head_body,
        grid=(N_HUM,),
        in_specs=[
            pl.BlockSpec((N_DET, C * NPTS), lambda h: (h, 0)),
            full((N_DET, 1)),
            full((N_DET, 1)),
            full((1, NUM_OBJ)),
            full((C * NPTS, REP)),
            full((1, REP)),
            full((REP, REP)),
            full((1, REP)),
            full((REP, NUM_CLASSES)),
            full((1, NUM_CLASSES)),
        ],
        out_specs=pl.BlockSpec((N_DET, NUM_CLASSES), lambda h: (h, 0)),
        out_shape=jax.ShapeDtypeStruct((P, NUM_CLASSES), jnp.float32),
    )(x, scores_c, labels_f, o2t_f, W1p, b1r, W2, b2r, W3, b3r)


def kernel(features, boxes, scores, labels, W1, b1, W2, b2, W3, b3, obj2target):
    featT = features.transpose(1, 2, 0).reshape(FH * FW, C)
    W1p = W1.reshape(C, NPTS, REP).transpose(1, 0, 2).reshape(C * NPTS, REP)
    idx = _pair_indices(boxes)
    pooled = _sc_gather(featT, idx.reshape(1, NIDX))
    x = pooled.reshape(P, C * NPTS)
    return _head(
        x,
        scores.reshape(N_DET, 1),
        labels.astype(jnp.float32).reshape(N_DET, 1),
        obj2target.astype(jnp.float32).reshape(1, NUM_OBJ),
        W1p,
        b1.reshape(1, REP),
        W2,
        b2.reshape(1, REP),
        W3,
        b3.reshape(1, NUM_CLASSES),
    )


# speedup vs baseline: 1.0288x; 1.0288x over previous
import functools

import jax
import jax.numpy as jnp
from jax.experimental import pallas as pl
from jax.experimental.pallas import tpu as pltpu
from jax.experimental.pallas import tpu_sc as plsc

NUM_CLASSES = 117
NUM_OBJ = 80
N_DET = 1000
N_HUM = 16
POOL = 4
NPTS = POOL * POOL
C = 64
FH = FW = 64
THRESH = 0.2
P = N_HUM * N_DET
NIDX = P * NPTS
REP = 128
GATHER_WIN = 256


def _idx_body(boxes_ref, out_ref):
    h = pl.program_id(0)
    b = boxes_ref[...]
    rowi = jax.lax.broadcasted_iota(jnp.int32, (N_DET, 1), 0)
    hm = rowi == h
    bh = jnp.sum(jnp.where(hm, b, 0.0), axis=0, keepdims=True)
    x1h, y1h, x2h, y2h = (bh[:, 0:1], bh[:, 1:2], bh[:, 2:3], bh[:, 3:4])
    x1o, y1o, x2o, y2o = (b[:, 0:1], b[:, 1:2], b[:, 2:3], b[:, 3:4])
    ux1 = jnp.minimum(x1h, x1o)
    uy1 = jnp.minimum(y1h, y1o)
    ux2 = jnp.maximum(x2h, x2o)
    uy2 = jnp.maximum(y2h, y2o)
    f = (jax.lax.broadcasted_iota(jnp.int32, (1, POOL), 1).astype(jnp.float32)
         + 0.5) / POOL
    gx = ux1 + (ux2 - ux1) * f
    gy = uy1 + (uy2 - uy1) * f
    xi = jnp.clip(jnp.round(gx), 0.0, FW - 1)
    yi = jnp.clip(jnp.round(gy), 0.0, FH - 1)
    idx = jnp.concatenate(
        [yi[:, i : i + 1] * FW + xi for i in range(POOL)], axis=1
    )
    out_ref[...] = idx.astype(jnp.int32)


def _pair_indices(boxes):
    return pl.pallas_call(
        _idx_body,
        grid=(N_HUM,),
        in_specs=[pl.BlockSpec((N_DET, 4), lambda h: (0, 0))],
        out_specs=pl.BlockSpec((N_DET, NPTS), lambda h: (h, 0)),
        out_shape=jax.ShapeDtypeStruct((P, NPTS), jnp.int32),
    )(boxes)


def _sc_gather(featT, idx_flat):
    mesh = plsc.VectorSubcoreMesh(core_axis_name="c", subcore_axis_name="s")

    @functools.partial(
        pl.kernel,
        out_type=jax.ShapeDtypeStruct((NIDX, C), jnp.float32),
        mesh=mesh,
        compiler_params=pltpu.CompilerParams(use_tc_tiling_on_sc=False),
    )
    def gk(x_hbm, i_hbm, o_hbm):
        def body(i_vmem, o_vmem):
            pltpu.sync_copy(x_hbm.at[i_vmem.at[0]], o_vmem)

        nchunks = NIDX // GATHER_WIN
        pltpu.emit_pipeline(
            body,
            grid=(2, nchunks // 2),
            in_specs=[
                pl.BlockSpec(
                    (1, GATHER_WIN),
                    lambda i, j: (0, i * (nchunks // 2) + j),
                )
            ],
            out_specs=[
                pl.BlockSpec(
                    (GATHER_WIN, C),
                    lambda i, j: (i * (nchunks // 2) + j, 0),
                )
            ],
            core_axis_name=("c", "s"),
            dimension_semantics=(pltpu.PARALLEL, pltpu.PARALLEL),
        )(i_hbm, o_hbm)

    return gk(featT, idx_flat)


def _head_body(x_ref, sc_ref, lab_ref, o2t_ref, w1_ref, b1_ref, w2_ref,
               b2_ref, w3_ref, b3_ref, out_ref):
    h = pl.program_id(0)
    rowi = jax.lax.broadcasted_iota(jnp.int32, (N_DET, 1), 0)
    hm = rowi == h
    s = sc_ref[...]
    se = jnp.where(s >= THRESH, s, 0.0)
    sh = jnp.sum(jnp.where(hm, se, 0.0))
    ds = sh * se * jnp.where(hm, 0.0, 1.0)
    lab = lab_ref[...]
    l_iota = jax.lax.broadcasted_iota(jnp.int32, (N_DET, NUM_OBJ), 1).astype(
        jnp.float32)
    ohl = jnp.where(lab == l_iota, 1.0, 0.0)
    tgt = jnp.sum(ohl * o2t_ref[...], axis=1, keepdims=True)

    x = x_ref[...]
    h1 = jax.nn.relu(
        jnp.dot(x, w1_ref[...], preferred_element_type=jnp.float32)
        + b1_ref[...]
    )
    h2 = jax.nn.relu(
        jnp.dot(h1, w2_ref[...], preferred_element_type=jnp.float32)
        + b2_ref[...]
    )
    logits = (
        jnp.dot(h2, w3_ref[...], preferred_element_type=jnp.float32)
        + b3_ref[...]
    )
    k_iota = jax.lax.broadcasted_iota(jnp.int32, (N_DET, NUM_CLASSES), 1
                                      ).astype(jnp.float32)
    onehot = jnp.where(tgt == k_iota, 1.0, 0.0)
    out_ref[...] = ds * onehot * jax.nn.sigmoid(logits)


def _head(x, scores_c, labels_f, o2t_f, W1p, b1r, W2, b2r, W3, b3r):
    full = lambda shape: pl.BlockSpec(shape, lambda h: (0, 0))
    return pl.pallas_call(
        _head_body,
        grid=(N_HUM,),
        in_specs=[
            pl.BlockSpec((N_DET, C * NPTS), lambda h: (h, 0)),
            full((N_DET, 1)),
            full((N_DET, 1)),
            full((1, NUM_OBJ)),
            full((C * NPTS, REP)),
            full((1, REP)),
            full((REP, REP)),
            full((1, REP)),
            full((REP, NUM_CLASSES)),
            full((1, NUM_CLASSES)),
        ],
        out_specs=pl.BlockSpec((N_DET, NUM_CLASSES), lambda h: (h, 0)),
        out_shape=jax.ShapeDtypeStruct((P, NUM_CLASSES), jnp.float32),
    )(x, scores_c, labels_f, o2t_f, W1p, b1r, W2, b2r, W3, b3r)


def kernel(features, boxes, scores, labels, W1, b1, W2, b2, W3, b3, obj2target):
    featT = features.transpose(1, 2, 0).reshape(FH * FW, C)
    W1p = W1.reshape(C, NPTS, REP).transpose(1, 0, 2).reshape(C * NPTS, REP)
    idx = _pair_indices(boxes)
    pooled = _sc_gather(featT, idx.reshape(1, NIDX))
    x = pooled.reshape(P, C * NPTS)
    return _head(
        x,
        scores.reshape(N_DET, 1),
        labels.astype(jnp.float32).reshape(N_DET, 1),
        obj2target.astype(jnp.float32).reshape(1, NUM_OBJ),
        W1p,
        b1.reshape(1, REP),
        W2,
        b2.reshape(1, REP),
        W3,
        b3.reshape(1, NUM_CLASSES),
    )
